# baseline (device time: 21758 ns/iter reference)
import jax
import jax.numpy as jnp
from jax import lax
from jax.experimental import pallas as pl
from jax.experimental.pallas import tpu as pltpu

K = 16
NEG = float("-inf")
FOLD_LEVELS = 4
N_CAND = 81
NBLK = 4


def kernel(x):
    m, n = x.shape
    mb = m // NBLK

    def extract_topk(a, k):
        rows = a.shape[0]
        if k == 1:
            return jnp.max(a, axis=1, keepdims=True)
        kcol = lax.broadcasted_iota(jnp.int32, (rows, k), 1)
        vals = jnp.full((rows, k), NEG, jnp.float32)
        for r in range(k):
            cur = jnp.max(a, axis=1, keepdims=True)
            vals = jnp.where(kcol == r, cur, vals)
            if r < k - 1:
                a = jnp.where(a == cur, NEG, a)
        return vals

    def extract_topk_batched(arrs, k):
        if k == 1:
            return [jnp.max(a, axis=1, keepdims=True) for a in arrs]
        if len(arrs) == 1:
            return [extract_topk(arrs[0], k)]
        rows = arrs[0].shape[0]
        b = jnp.stack(arrs, axis=0)
        c = len(arrs)
        kcol = lax.broadcasted_iota(jnp.int32, (c, rows, k), 2)
        vals = jnp.full((c, rows, k), NEG, jnp.float32)
        for r in range(k):
            cur = jnp.max(b, axis=2, keepdims=True)
            vals = jnp.where(kcol == r, cur, vals)
            if r < k - 1:
                b = jnp.where(b == cur, NEG, b)
        return [vals[i] for i in range(c)]

    def local_topk_cands(xblk):
        jobs = [(xblk, K)]
        for _ in range(FOLD_LEVELS):
            nxt = []
            for a, k in jobs:
                h = a.shape[1] // 2
                lo, hi = a[:, :h], a[:, h:]
                nxt.append((jnp.maximum(lo, hi), k))
                if k >= 2:
                    nxt.append((jnp.minimum(lo, hi), k // 2))
            jobs = nxt
        by_k: dict = {}
        for a, k in jobs:
            by_k.setdefault(k, []).append(a)
        pieces = []
        for k in sorted(by_k, reverse=True):
            pieces.extend(extract_topk_batched(by_k[k], k))
        return jnp.concatenate(pieces, axis=1)

    def body(x_ref, out_ref, xb, send_buf, recv_buf,
             copy_sems, send_sems, recv_sems):
        my_x = lax.axis_index("x")
        my_y = lax.axis_index("y")
        my_z = lax.axis_index("z")
        nbr = (my_x, 1 - my_y, my_z)

        def in_copy(b):
            return pltpu.make_async_copy(
                x_ref.at[pl.ds(b * mb, mb), :], xb.at[b], copy_sems.at[b]
            )

        def rdma(b):
            return pltpu.make_async_remote_copy(
                src_ref=send_buf.at[b],
                dst_ref=recv_buf.at[b],
                send_sem=send_sems.at[b],
                recv_sem=recv_sems.at[b],
                device_id=nbr,
                device_id_type=pl.DeviceIdType.MESH,
            )

        for b in range(NBLK):
            in_copy(b).start()

        barrier_sem = pltpu.get_barrier_semaphore()
        pl.semaphore_signal(
            barrier_sem, inc=1, device_id=nbr,
            device_id_type=pl.DeviceIdType.MESH,
        )
        pl.semaphore_wait(barrier_sem, 1)

        cands = []
        for b in range(NBLK):
            in_copy(b).wait()
            cb = local_topk_cands(xb[b])
            cands.append(cb)
            send_buf[b] = cb
            rdma(b).start()

        for b in range(NBLK):
            rdma(b).wait_recv()
            allc = jnp.concatenate([cands[b], recv_buf[b]], axis=1)
            out_ref[pl.ds(b * mb, mb), :] = extract_topk(allc, K)

        for b in range(NBLK):
            rdma(b).wait_send()

    return pl.pallas_call(
        body,
        out_shape=jax.ShapeDtypeStruct((m, K), jnp.float32),
        in_specs=[pl.BlockSpec(memory_space=pltpu.MemorySpace.HBM)],
        out_specs=pl.BlockSpec(memory_space=pltpu.VMEM),
        scratch_shapes=[
            pltpu.VMEM((NBLK, mb, n), jnp.float32),
            pltpu.VMEM((NBLK, mb, N_CAND), jnp.float32),
            pltpu.VMEM((NBLK, mb, N_CAND), jnp.float32),
            pltpu.SemaphoreType.DMA((NBLK,)),
            pltpu.SemaphoreType.DMA((NBLK,)),
            pltpu.SemaphoreType.DMA((NBLK,)),
        ],
        compiler_params=pltpu.CompilerParams(collective_id=0),
    )(x)
